# baseline (speedup 1.0000x reference)
import functools

import jax
import jax.numpy as jnp
from jax import lax
from jax.experimental import pallas as pl
from jax.experimental.pallas import tpu as pltpu
from jax.experimental.pallas import tpu_sc as plsc

B = 1024
L = 200
D = 64
NC = 2
NS = 16
NW = NC * NS
BPW = B // NW
S0 = 104
S1 = L - S0
NBUF = 3


def _sc_embed(dx_ints, proc_ints, dx_table, proc_table, pe):
    mesh = plsc.VectorSubcoreMesh(core_axis_name="c", subcore_axis_name="s")

    scratch = {
        "idx_dx": pltpu.VMEM((BPW, L), jnp.int32),
        "idx_pr": pltpu.VMEM((BPW, L), jnp.int32),
        "pe_v": pltpu.VMEM((L, 1, D), jnp.float32),
        "rows": [pltpu.VMEM((L, D), jnp.float32) for _ in range(2 * NBUF)],
        "gsem": [pltpu.SemaphoreType.DMA for _ in range(NBUF)],
        "ssem": [pltpu.SemaphoreType.DMA for _ in range(NBUF)],
    }

    @functools.partial(
        pl.kernel,
        out_type=(
            jax.ShapeDtypeStruct((B, L, D), jnp.float32),
            jax.ShapeDtypeStruct((B, L, D), jnp.float32),
        ),
        mesh=mesh,
        compiler_params=pltpu.CompilerParams(use_tc_tiling_on_sc=False),
        scratch_types=scratch,
    )
    def k(dxi_hbm, pri_hbm, dxt_hbm, prt_hbm, pe_hbm, dx_out, pr_out,
          idx_dx, idx_pr, pe_v, rows, gsem, ssem):
        wid = lax.axis_index("s") * NC + lax.axis_index("c")
        pltpu.sync_copy(pe_hbm.at[pl.ds(0, L)], pe_v)
        pltpu.sync_copy(dxi_hbm.at[pl.ds(wid * BPW, BPW)], idx_dx)
        pltpu.sync_copy(pri_hbm.at[pl.ds(wid * BPW, BPW)], idx_pr)

        def fire_gathers(i):
            s = i % NBUF
            return [
                pltpu.async_copy(dxt_hbm.at[idx_dx.at[i, pl.ds(0, S0)]],
                                 rows[2 * s].at[pl.ds(0, S0)], gsem[s]),
                pltpu.async_copy(dxt_hbm.at[idx_dx.at[i, pl.ds(S0, S1)]],
                                 rows[2 * s].at[pl.ds(S0, S1)], gsem[s]),
                pltpu.async_copy(prt_hbm.at[idx_pr.at[i, pl.ds(0, S0)]],
                                 rows[2 * s + 1].at[pl.ds(0, S0)], gsem[s]),
                pltpu.async_copy(prt_hbm.at[idx_pr.at[i, pl.ds(S0, S1)]],
                                 rows[2 * s + 1].at[pl.ds(S0, S1)], gsem[s]),
            ]

        def add_pe(buf):
            @plsc.parallel_loop(0, L, unroll=4)
            def _(j):
                for q in range(D // 16):
                    sl = pl.ds(q * 16, 16)
                    plsc.addupdate(buf.at[j, sl], pe_v[j, 0, sl])

        gd = [None] * BPW
        sd = [None] * BPW
        gd[0] = fire_gathers(0)
        gd[1] = fire_gathers(1)
        for i in range(BPW):
            s = i % NBUF
            b = wid * BPW + i
            for g in gd[i]:
                g.wait()
            add_pe(rows[2 * s])
            st0 = pltpu.async_copy(rows[2 * s], dx_out.at[b], ssem[s])
            add_pe(rows[2 * s + 1])
            st1 = pltpu.async_copy(rows[2 * s + 1], pr_out.at[b], ssem[s])
            sd[i] = (st0, st1)
            if i + 2 < BPW:
                if i >= 1:
                    for st in sd[i - 1]:
                        st.wait()
                gd[i + 2] = fire_gathers(i + 2)
        for i in range(BPW - NBUF, BPW):
            for st in sd[i]:
                st.wait()

    return k(dx_ints, proc_ints, dx_table, proc_table, pe)


def kernel(dx_ints, proc_ints, dx_table, proc_table, visit_table, pe):
    dx_emb, proc_emb = _sc_embed(dx_ints.astype(jnp.int32),
                                 proc_ints.astype(jnp.int32),
                                 dx_table, proc_table, pe)
    visit = jnp.broadcast_to(visit_table[0][None, None, :], (B, 1, D))
    visit_mask = jnp.ones((B, 1), dtype=jnp.float32)
    return (dx_emb, proc_emb, visit, visit_mask)


# speedup vs baseline: 2.3153x; 2.3153x over previous
import functools

import jax
import jax.numpy as jnp
from jax import lax
from jax.experimental import pallas as pl
from jax.experimental.pallas import tpu as pltpu
from jax.experimental.pallas import tpu_sc as plsc

B = 1024
L = 200
D = 64
NC = 2
NS = 16
NW = NC * NS
BC = 256
UPW = 800 // 16
NLR = 13


def _sc_embed(dxi_t, pri_t, dx_table, proc_table, pe_block):
    mesh = plsc.VectorSubcoreMesh(core_axis_name="c", subcore_axis_name="s")

    scratch = {
        "idx_v": pltpu.VMEM((3, 8, 8, 128), jnp.int32),
        "pe_v": pltpu.VMEM((NLR, D), jnp.float32),
        "gbuf": [pltpu.VMEM((BC, D), jnp.float32) for _ in range(2)],
        "sbuf": [pltpu.VMEM((8, 2, 12, 129), jnp.float32) for _ in range(2)],
        "gsem": [pltpu.SemaphoreType.DMA for _ in range(2)],
        "ssem": [pltpu.SemaphoreType.DMA for _ in range(2)],
    }

    @functools.partial(
        pl.kernel,
        out_type=(
            jax.ShapeDtypeStruct((L, 8, 8, 8, 128), jnp.float32),
            jax.ShapeDtypeStruct((L, 8, 8, 8, 128), jnp.float32),
        ),
        mesh=mesh,
        compiler_params=pltpu.CompilerParams(use_tc_tiling_on_sc=False,
                                             needs_layout_passes=False),
        scratch_types=scratch,
    )
    def k(dxi_hbm, pri_hbm, dxt_hbm, prt_hbm, pe_hbm, dx_out, pr_out,
          idx_v, pe_v, gbuf, sbuf, gsem, ssem):
        wid = lax.axis_index("s") * NC + lax.axis_index("c")
        wloc = lax.rem(wid, 16)
        l0 = (UPW * wloc) // 4
        tr0 = jnp.minimum(l0 // 8, 25 - 3)
        pltpu.sync_copy(pe_hbm.at[pl.ds(l0, NLR)], pe_v)

        def run(idx_hbm, tbl_hbm, out_hbm):
            pltpu.sync_copy(idx_hbm.at[pl.ds(tr0, 3)], idx_v)
            dl = lax.iota(jnp.int32, 16)
            dts = [2 * dg + dl // 8 for dg in range(4)]
            din = lax.rem(dl, 8)
            btq_v = [jnp.full((16,), btq, jnp.int32) for btq in range(2)]

            def fire(k_unit, p):
                u = UPW * wloc + k_unit
                bq = lax.rem(u, 4)
                l = u // 4
                tr_rel = l // 8 - tr0
                r = lax.rem(l, 8)
                pltpu.async_copy(
                    tbl_hbm.at[idx_v.at[tr_rel, 2 * bq, r]],
                    gbuf[p].at[pl.ds(0, 128)], gsem[p])
                pltpu.async_copy(
                    tbl_hbm.at[idx_v.at[tr_rel, 2 * bq + 1, r]],
                    gbuf[p].at[pl.ds(128, 128)], gsem[p])

            def wait_g(p):
                pltpu.make_async_copy(
                    tbl_hbm.at[pl.ds(0, BC)], gbuf[p], gsem[p]).wait()

            def wait_s(p):
                pltpu.make_async_copy(
                    sbuf[p].at[:, :, pl.ds(0, 8), pl.ds(0, 128)],
                    out_hbm.at[0, :, pl.ds(0, 2)], ssem[p]).wait()

            def unit(k_unit, h, p):
                u = UPW * wloc + k_unit
                l_rel = u // 4 - l0
                bq = lax.rem(u, 4)
                wait_g(p)

                @pl.when(h >= 1)
                def _():
                    wait_s(p)

                pe16 = [pe_v[l_rel, pl.ds(dg * 16, 16)] for dg in range(4)]
                for btq in range(2):
                    @plsc.parallel_loop(0, 128, unroll=4)
                    def _(bin_, btq=btq):
                        bin_v = jnp.full((16,), bin_, jnp.int32)
                        bloc = btq * 128 + bin_
                        for dg in range(4):
                            val = (gbuf[p][bloc, pl.ds(dg * 16, 16)]
                                   + pe16[dg])
                            plsc.store_scatter(
                                sbuf[p], [dts[dg], btq_v[btq], din, bin_v],
                                val)

                pltpu.async_copy(
                    sbuf[p].at[:, :, pl.ds(0, 8), pl.ds(0, 128)],
                    out_hbm.at[u // 4, :, pl.ds(2 * bq, 2)], ssem[p])

                @pl.when(k_unit + 2 < UPW)
                def _():
                    fire(k_unit + 2, p)

            fire(0, 0)
            fire(1, 1)

            def body(h, carry):
                unit(2 * h, h, 0)
                unit(2 * h + 1, h, 1)
                return carry

            lax.fori_loop(0, UPW // 2, body, 0)
            wait_s(0)
            wait_s(1)

        @pl.when(wid < 16)
        def _():
            run(dxi_hbm, dxt_hbm, dx_out)

        @pl.when(wid >= 16)
        def _():
            run(pri_hbm, prt_hbm, pr_out)

    return k(dxi_t, pri_t, dx_table, proc_table, pe_block)


def kernel(dx_ints, proc_ints, dx_table, proc_table, visit_table, pe):
    dxi_t = (jnp.transpose(dx_ints.astype(jnp.int32))
             .reshape(25, 8, 8, 128).transpose(0, 2, 1, 3))
    pri_t = (jnp.transpose(proc_ints.astype(jnp.int32))
             .reshape(25, 8, 8, 128).transpose(0, 2, 1, 3))
    pe_block = pe[:L, 0, :]
    o1, o2 = _sc_embed(dxi_t, pri_t, dx_table, proc_table, pe_block)
    dx_emb = o1.transpose(2, 4, 0, 1, 3).reshape(B, L, D)
    proc_emb = o2.transpose(2, 4, 0, 1, 3).reshape(B, L, D)
    visit = jnp.broadcast_to(visit_table[0][None, None, :], (B, 1, D))
    visit_mask = jnp.ones((B, 1), dtype=jnp.float32)
    return (dx_emb, proc_emb, visit, visit_mask)
